# Initial kernel scaffold; baseline (speedup 1.0000x reference)
import jax
import jax.numpy as jnp
from jax.experimental import pallas as pl


def kernel(x, tW1, tb1, tW2, tb2, tW3, tb3, sW1, sb1, sW2, sb2):
    raise NotImplementedError("write your pallas kernel here")


# speedup vs baseline: 19.4872x; 19.4872x over previous
import jax
import jax.numpy as jnp
from jax.experimental import pallas as pl
from jax.experimental.pallas import tpu as pltpu

F32 = jnp.float32
BF16 = jnp.bfloat16
_B, _C, _T, _K, _G = 2, 256, 2048, 10, 32
_W = 128
_RT = 256
_NS = 16
_TN = 512
_DP = jax.lax.Precision.DEFAULT


def _mm(a, b, prec=_DP):
    return jax.lax.dot_general(a, b, (((1,), (0,)), ((), ())),
                               precision=prec, preferred_element_type=F32)


def _prep_kernel(xt_ref, tW1T_ref, tb1_ref, W2blkT_ref, tb2_ref, tW3T_ref,
                 tb3_ref, W1aT_ref, W1bT_ref, sb1_ref,
                 toutT_ref, At_ref, BvT_ref):
    xt = xt_ref[0]
    t1 = jnp.maximum(_mm(xt, tW1T_ref[...]) + tb1_ref[...], 0.0)
    z = jnp.zeros((1, _W), F32)
    t1m = jnp.concatenate([z, t1[:-1]], axis=0)
    t1p = jnp.concatenate([t1[1:], z], axis=0)
    t2 = (_mm(t1m, W2blkT_ref[0]) + _mm(t1, W2blkT_ref[1])
          + _mm(t1p, W2blkT_ref[2]))
    t2 = jnp.maximum(t2 + tb2_ref[...], 0.0)
    toutT_ref[0] = _mm(t2, tW3T_ref[...]) + tb3_ref[...]
    At_ref[0] = _mm(xt, W1aT_ref[...]).astype(BF16)
    BvT_ref[0] = _mm(xt, W1bT_ref[...]) + sb1_ref[...]


def _knn_kernel(xt_tile_ref, xt_full_ref, x_ref, At_ref, BvT_ref, pre_ref):
    xt_t = xt_tile_ref[0]
    xt_f = xt_full_ref[0]
    g = jax.lax.dot_general(xt_t, xt_f, (((1,), (1,)), ((), ())),
                            precision=_DP, preferred_element_type=F32)
    xx_t = jnp.sum(xt_t * xt_t, axis=1, keepdims=True)
    xb = x_ref[0]
    xx_f = jnp.sum(xb * xb, axis=0, keepdims=True)
    d = 2.0 * g - xx_t - xx_f
    iota = jax.lax.broadcasted_iota(jnp.int32, (_RT, _T), 1)
    at = At_ref[0]
    bv = BvT_ref[0]
    for i in range(_K):
        m = jnp.max(d, axis=1, keepdims=True)
        cand = jnp.where(d == m, iota, _T)
        am = jnp.min(cand, axis=1, keepdims=True)
        oh = iota == am
        f = jax.lax.dot_general(oh.astype(BF16), at, (((1,), (0,)), ((), ())),
                                preferred_element_type=F32)
        y = f + bv
        pre = 0.5 * y * (1.0 + jax.lax.erf(y * 0.7071067811865476))
        pre_ref[0, :, (3 + i) * _W:(4 + i) * _W] = pre.astype(BF16)
        d = jnp.where(oh, -jnp.inf, d)
    zeros = jnp.zeros((_RT, _W), BF16)
    for s in (0, 1, 2, 13, 14, 15):
        pre_ref[0, :, s * _W:(s + 1) * _W] = zeros


def _conv_kernel(pre_ref, toutT_ref, xt_ref, W2Tb_ref, sb2_ref, out_ref):
    smax = jnp.full((_TN, _C), -jnp.inf, F32)
    for j in range(_K):
        acc = jnp.zeros((_TN, _C), F32)
        for dh in range(7):
            xin = pre_ref[0, dh, :, j * _W:(j + 7) * _W]
            acc = acc + jax.lax.dot_general(
                xin, W2Tb_ref[dh], (((1,), (0,)), ((), ())),
                preferred_element_type=F32)
        smax = jnp.maximum(smax, acc)
    outv = toutT_ref[0] + xt_ref[0] + smax + sb2_ref[...]
    out_ref[0] = jnp.maximum(outv, 0.0)


def kernel(x, tW1, tb1, tW2, tb2, tW3, tb3, sW1, sb1, sW2, sb2):
    xt = jnp.transpose(x, (0, 2, 1))

    eye = jnp.eye(_G, dtype=F32)
    tmp = tW2.reshape(_G, 4, 4, 3)
    W2blkT = jnp.einsum('gh,hoid->dgiho', eye, tmp).reshape(3, _W, _W)
    tW1T = tW1[:, :, 0].T
    tW3T = tW3[:, :, 0].T
    W1aT = sW1[:, :_C, 0, 0].T
    W1bT = sW1[:, _C:, 0, 0].T
    tmp2 = sW2.reshape(_G, 8, 4, 7, 7)
    W2T = jnp.einsum('gh,hoidw->dwgiho', eye, tmp2).reshape(7, 7, _W, _C)
    W2Tb = W2T.transpose(0, 1, 2, 3).reshape(7, 7 * _W, _C).astype(BF16)
    tb1r = tb1.reshape(1, _W)
    tb2r = tb2.reshape(1, _W)
    tb3r = tb3.reshape(1, _C)
    sb1r = sb1.reshape(1, _W)
    sb2r = sb2.reshape(1, _C)

    toutT, At, BvT = pl.pallas_call(
        _prep_kernel,
        grid=(_B,),
        in_specs=[
            pl.BlockSpec((1, _T, _C), lambda b: (b, 0, 0)),
            pl.BlockSpec((_C, _W), lambda b: (0, 0)),
            pl.BlockSpec((1, _W), lambda b: (0, 0)),
            pl.BlockSpec((3, _W, _W), lambda b: (0, 0, 0)),
            pl.BlockSpec((1, _W), lambda b: (0, 0)),
            pl.BlockSpec((_W, _C), lambda b: (0, 0)),
            pl.BlockSpec((1, _C), lambda b: (0, 0)),
            pl.BlockSpec((_C, _W), lambda b: (0, 0)),
            pl.BlockSpec((_C, _W), lambda b: (0, 0)),
            pl.BlockSpec((1, _W), lambda b: (0, 0)),
        ],
        out_specs=[
            pl.BlockSpec((1, _T, _C), lambda b: (b, 0, 0)),
            pl.BlockSpec((1, _T, _W), lambda b: (b, 0, 0)),
            pl.BlockSpec((1, _T, _W), lambda b: (b, 0, 0)),
        ],
        out_shape=[
            jax.ShapeDtypeStruct((_B, _T, _C), F32),
            jax.ShapeDtypeStruct((_B, _T, _W), BF16),
            jax.ShapeDtypeStruct((_B, _T, _W), F32),
        ],
    )(xt, tW1T, tb1r, W2blkT, tb2r, tW3T, tb3r, W1aT, W1bT, sb1r)

    nrt = _T // _RT
    pre = pl.pallas_call(
        _knn_kernel,
        grid=(_B, nrt),
        in_specs=[
            pl.BlockSpec((1, _RT, _C), lambda b, r: (b, r, 0)),
            pl.BlockSpec((1, _T, _C), lambda b, r: (b, 0, 0)),
            pl.BlockSpec((1, _C, _T), lambda b, r: (b, 0, 0)),
            pl.BlockSpec((1, _T, _W), lambda b, r: (b, 0, 0)),
            pl.BlockSpec((1, _RT, _W), lambda b, r: (b, r, 0)),
        ],
        out_specs=pl.BlockSpec((1, _RT, _NS * _W), lambda b, r: (b, r, 0)),
        out_shape=jax.ShapeDtypeStruct((_B, _T, _NS * _W), BF16),
    )(xt, xt, x, At, BvT)

    pre_z = jnp.pad(pre, ((0, 0), (3, 3), (0, 0)))
    pre_sh = jnp.stack([pre_z[:, d:d + _T] for d in range(7)], axis=1)

    ntn = _T // _TN
    outT = pl.pallas_call(
        _conv_kernel,
        grid=(_B, ntn),
        in_specs=[
            pl.BlockSpec((1, 7, _TN, _NS * _W), lambda b, n: (b, 0, n, 0)),
            pl.BlockSpec((1, _TN, _C), lambda b, n: (b, n, 0)),
            pl.BlockSpec((1, _TN, _C), lambda b, n: (b, n, 0)),
            pl.BlockSpec((7, 7 * _W, _C), lambda b, n: (0, 0, 0)),
            pl.BlockSpec((1, _C), lambda b, n: (0, 0)),
        ],
        out_specs=pl.BlockSpec((1, _TN, _C), lambda b, n: (b, n, 0)),
        out_shape=jax.ShapeDtypeStruct((_B, _T, _C), F32),
    )(pre_sh, toutT, xt, W2Tb, sb2r)

    return jnp.transpose(outT, (0, 2, 1))


# speedup vs baseline: 23.7903x; 1.2208x over previous
import jax
import jax.numpy as jnp
from jax.experimental import pallas as pl
from jax.experimental.pallas import tpu as pltpu

F32 = jnp.float32
BF16 = jnp.bfloat16
_B, _C, _T, _K, _G = 2, 256, 2048, 10, 32
_W = 128
_RT = 256
_NS = 16
_TN = 512
_DP = jax.lax.Precision.DEFAULT


def _mm(a, b, prec=_DP):
    return jax.lax.dot_general(a, b, (((1,), (0,)), ((), ())),
                               precision=prec, preferred_element_type=F32)


def _prep_kernel(xt_ref, tW1T_ref, tb1_ref, W2blkT_ref, tb2_ref, tW3T_ref,
                 tb3_ref, W1aT_ref, W1bT_ref, sb1_ref,
                 toutT_ref, At_ref, BvT_ref):
    xt = xt_ref[0]
    t1 = jnp.maximum(_mm(xt, tW1T_ref[...]) + tb1_ref[...], 0.0)
    z = jnp.zeros((1, _W), F32)
    t1m = jnp.concatenate([z, t1[:-1]], axis=0)
    t1p = jnp.concatenate([t1[1:], z], axis=0)
    t2 = (_mm(t1m, W2blkT_ref[0]) + _mm(t1, W2blkT_ref[1])
          + _mm(t1p, W2blkT_ref[2]))
    t2 = jnp.maximum(t2 + tb2_ref[...], 0.0)
    toutT_ref[0] = _mm(t2, tW3T_ref[...]) + tb3_ref[...]
    At_ref[0] = _mm(xt, W1aT_ref[...]).astype(BF16)
    BvT_ref[0] = _mm(xt, W1bT_ref[...]) + sb1_ref[...]


def _knn_kernel(xt_tile_ref, xt_full_ref, x_ref, At_ref, BvT_ref, pre_ref):
    xt_t = xt_tile_ref[0]
    xt_f = xt_full_ref[0]
    g = jax.lax.dot_general(xt_t, xt_f, (((1,), (1,)), ((), ())),
                            precision=_DP, preferred_element_type=F32)
    xx_t = jnp.sum(xt_t * xt_t, axis=1, keepdims=True)
    xb = x_ref[0]
    xx_f = jnp.sum(xb * xb, axis=0, keepdims=True)
    d = 2.0 * g - xx_t - xx_f
    iota = jax.lax.broadcasted_iota(jnp.int32, (_RT, _T), 1)
    at = At_ref[0]
    bv = BvT_ref[0]
    for i in range(_K):
        m = jnp.max(d, axis=1, keepdims=True)
        cand = jnp.where(d == m, iota, _T)
        am = jnp.min(cand, axis=1, keepdims=True)
        oh = iota == am
        f = jax.lax.dot_general(oh.astype(BF16), at, (((1,), (0,)), ((), ())),
                                preferred_element_type=F32)
        y = f + bv
        pre = 0.5 * y * (1.0 + jax.lax.erf(y * 0.7071067811865476))
        pre_ref[0, :, (3 + i) * _W:(4 + i) * _W] = pre.astype(BF16)
        d = jnp.where(oh, -jnp.inf, d)
    zeros = jnp.zeros((_RT, _W), BF16)
    for s in (0, 1, 2, 13, 14, 15):
        pre_ref[0, :, s * _W:(s + 1) * _W] = zeros


def _conv_kernel(pre_ref, toutT_ref, xt_ref, W2Tb_ref, sb2_ref, out_ref):
    nt = pl.program_id(1)
    win = pre_ref[0, pl.ds(nt * _TN, _TN + 8), :]
    smax = jnp.full((_TN, _C), -jnp.inf, F32)
    for j in range(_K):
        lo = max(0, 3 - j)
        hi = min(7, 13 - j)
        acc = jnp.zeros((_TN, _C), F32)
        for dh in range(7):
            xin = win[dh + 1:dh + 1 + _TN, (j + lo) * _W:(j + hi) * _W]
            acc = acc + jax.lax.dot_general(
                xin, W2Tb_ref[dh, lo * _W:hi * _W, :],
                (((1,), (0,)), ((), ())),
                preferred_element_type=F32)
        smax = jnp.maximum(smax, acc)
    outv = toutT_ref[0] + xt_ref[0] + smax + sb2_ref[...]
    out_ref[0] = jnp.maximum(outv, 0.0)


def kernel(x, tW1, tb1, tW2, tb2, tW3, tb3, sW1, sb1, sW2, sb2):
    xt = jnp.transpose(x, (0, 2, 1))

    eye = jnp.eye(_G, dtype=F32)
    tmp = tW2.reshape(_G, 4, 4, 3)
    W2blkT = jnp.einsum('gh,hoid->dgiho', eye, tmp).reshape(3, _W, _W)
    tW1T = tW1[:, :, 0].T
    tW3T = tW3[:, :, 0].T
    W1aT = sW1[:, :_C, 0, 0].T
    W1bT = sW1[:, _C:, 0, 0].T
    tmp2 = sW2.reshape(_G, 8, 4, 7, 7)
    W2T = jnp.einsum('gh,hoidw->dwgiho', eye, tmp2).reshape(7, 7, _W, _C)
    W2Tb = W2T.transpose(0, 1, 2, 3).reshape(7, 7 * _W, _C).astype(BF16)
    tb1r = tb1.reshape(1, _W)
    tb2r = tb2.reshape(1, _W)
    tb3r = tb3.reshape(1, _C)
    sb1r = sb1.reshape(1, _W)
    sb2r = sb2.reshape(1, _C)

    toutT, At, BvT = pl.pallas_call(
        _prep_kernel,
        grid=(_B,),
        in_specs=[
            pl.BlockSpec((1, _T, _C), lambda b: (b, 0, 0)),
            pl.BlockSpec((_C, _W), lambda b: (0, 0)),
            pl.BlockSpec((1, _W), lambda b: (0, 0)),
            pl.BlockSpec((3, _W, _W), lambda b: (0, 0, 0)),
            pl.BlockSpec((1, _W), lambda b: (0, 0)),
            pl.BlockSpec((_W, _C), lambda b: (0, 0)),
            pl.BlockSpec((1, _C), lambda b: (0, 0)),
            pl.BlockSpec((_C, _W), lambda b: (0, 0)),
            pl.BlockSpec((_C, _W), lambda b: (0, 0)),
            pl.BlockSpec((1, _W), lambda b: (0, 0)),
        ],
        out_specs=[
            pl.BlockSpec((1, _T, _C), lambda b: (b, 0, 0)),
            pl.BlockSpec((1, _T, _W), lambda b: (b, 0, 0)),
            pl.BlockSpec((1, _T, _W), lambda b: (b, 0, 0)),
        ],
        out_shape=[
            jax.ShapeDtypeStruct((_B, _T, _C), F32),
            jax.ShapeDtypeStruct((_B, _T, _W), BF16),
            jax.ShapeDtypeStruct((_B, _T, _W), F32),
        ],
    )(xt, tW1T, tb1r, W2blkT, tb2r, tW3T, tb3r, W1aT, W1bT, sb1r)

    nrt = _T // _RT
    pre = pl.pallas_call(
        _knn_kernel,
        grid=(_B, nrt),
        in_specs=[
            pl.BlockSpec((1, _RT, _C), lambda b, r: (b, r, 0)),
            pl.BlockSpec((1, _T, _C), lambda b, r: (b, 0, 0)),
            pl.BlockSpec((1, _C, _T), lambda b, r: (b, 0, 0)),
            pl.BlockSpec((1, _T, _W), lambda b, r: (b, 0, 0)),
            pl.BlockSpec((1, _RT, _W), lambda b, r: (b, r, 0)),
        ],
        out_specs=pl.BlockSpec((1, _RT, _NS * _W), lambda b, r: (b, r, 0)),
        out_shape=jax.ShapeDtypeStruct((_B, _T, _NS * _W), BF16),
    )(xt, xt, x, At, BvT)

    pre_z = jnp.pad(pre, ((0, 0), (4, 4), (0, 0)))

    ntn = _T // _TN
    outT = pl.pallas_call(
        _conv_kernel,
        grid=(_B, ntn),
        in_specs=[
            pl.BlockSpec((1, _T + 8, _NS * _W), lambda b, n: (b, 0, 0)),
            pl.BlockSpec((1, _TN, _C), lambda b, n: (b, n, 0)),
            pl.BlockSpec((1, _TN, _C), lambda b, n: (b, n, 0)),
            pl.BlockSpec((7, 7 * _W, _C), lambda b, n: (0, 0, 0)),
            pl.BlockSpec((1, _C), lambda b, n: (0, 0)),
        ],
        out_specs=pl.BlockSpec((1, _TN, _C), lambda b, n: (b, n, 0)),
        out_shape=jax.ShapeDtypeStruct((_B, _T, _C), F32),
    )(pre_z, toutT, xt, W2Tb, sb2r)

    return jnp.transpose(outT, (0, 2, 1))


# speedup vs baseline: 25.6713x; 1.0791x over previous
import jax
import jax.numpy as jnp
from jax.experimental import pallas as pl
from jax.experimental.pallas import tpu as pltpu

F32 = jnp.float32
BF16 = jnp.bfloat16
_B, _C, _T, _K, _G = 2, 256, 2048, 10, 32
_W = 128
_RT = 256
_NS = 16
_NT = _T // _RT
_PR = _T + 32
_DP = jax.lax.Precision.DEFAULT


def _mm(a, b):
    return jax.lax.dot_general(a, b, (((1,), (0,)), ((), ())),
                               precision=_DP, preferred_element_type=F32)


def _gelu(y):
    return 0.5 * y * (1.0 + jax.lax.erf(y * 0.7071067811865476))


def _fused_kernel(xt_tile_ref, xt_full_ref, x_ref,
                  tW1T_ref, tb1_ref, W2blkT_ref, tb2_ref, tW3T_ref, tb3_ref,
                  W1aT_ref, W1bT_ref, sb1_ref, W2Tb_ref, sb2_ref,
                  out_ref, pre_s, tout_s, at_s, xx_s):
    r = pl.program_id(1)

    @pl.when(r == 0)
    def _prep():
        xt = xt_full_ref[0]
        t1 = jnp.maximum(_mm(xt, tW1T_ref[...]) + tb1_ref[...], 0.0)
        z = jnp.zeros((1, _W), F32)
        t1m = jnp.concatenate([z, t1[:-1]], axis=0)
        t1p = jnp.concatenate([t1[1:], z], axis=0)
        t2 = (_mm(t1m, W2blkT_ref[0]) + _mm(t1, W2blkT_ref[1])
              + _mm(t1p, W2blkT_ref[2]))
        t2 = jnp.maximum(t2 + tb2_ref[...], 0.0)
        tout_s[...] = _mm(t2, tW3T_ref[...]) + tb3_ref[...]
        at_s[...] = _mm(xt, W1aT_ref[...]).astype(BF16)
        xb = x_ref[0]
        xx_s[0:1, :] = jnp.sum(xb * xb, axis=0, keepdims=True)
        zpad = jnp.zeros((16, _NS * _W), BF16)
        pre_s[0:16, :] = zpad
        pre_s[_PR - 16:_PR, :] = zpad

    @pl.when(r < _NT)
    def _knn():
        xt_t = xt_tile_ref[0]
        xt_f = xt_full_ref[0]
        g = jax.lax.dot_general(xt_t, xt_f, (((1,), (1,)), ((), ())),
                                precision=_DP, preferred_element_type=F32)
        xx_t = jnp.sum(xt_t * xt_t, axis=1, keepdims=True)
        d = 2.0 * g - xx_t - xx_s[0:1, :]
        iota = jax.lax.broadcasted_iota(jnp.int32, (_RT, _T), 1)
        rowid = r * _RT + jax.lax.broadcasted_iota(jnp.int32, (_RT, _T), 0)
        at = at_s[...]
        bv = _mm(xt_t, W1bT_ref[...]) + sb1_ref[...]
        row0 = pl.ds(r * _RT + 16, _RT)
        f0 = at_s[pl.ds(r * _RT, _RT), :].astype(F32)
        pre_s[row0, 3 * _W:4 * _W] = _gelu(f0 + bv).astype(BF16)
        d = jnp.where(iota == rowid, -jnp.inf, d)
        for i in range(1, _K):
            m = jnp.max(d, axis=1, keepdims=True)
            cand = jnp.where(d == m, iota, _T)
            am = jnp.min(cand, axis=1, keepdims=True)
            oh = cand == am
            f = jax.lax.dot_general(oh.astype(BF16), at,
                                    (((1,), (0,)), ((), ())),
                                    preferred_element_type=F32)
            pre_s[row0, (3 + i) * _W:(4 + i) * _W] = _gelu(f + bv).astype(BF16)
            d = jnp.where(oh, -jnp.inf, d)
        zeros = jnp.zeros((_RT, _W), BF16)
        for s in (0, 1, 2, 13, 14, 15):
            pre_s[row0, s * _W:(s + 1) * _W] = zeros

    @pl.when(r > 0)
    def _conv():
        c = r - 1
        win = pre_s[pl.ds(c * _RT, _RT + 32), :]
        smax = jnp.full((_RT, _C), -jnp.inf, F32)
        for j in range(_K):
            lo = max(0, 3 - j)
            hi = min(7, 13 - j)
            acc = jnp.zeros((_RT, _C), F32)
            for dh in range(7):
                xin = win[13 + dh:13 + dh + _RT, (j + lo) * _W:(j + hi) * _W]
                acc = acc + jax.lax.dot_general(
                    xin, W2Tb_ref[dh, lo * _W:hi * _W, :],
                    (((1,), (0,)), ((), ())),
                    preferred_element_type=F32)
            smax = jnp.maximum(smax, acc)
        tout_c = tout_s[pl.ds(c * _RT, _RT), :]
        xt_c = xt_full_ref[0, pl.ds(c * _RT, _RT), :]
        outv = tout_c + xt_c + smax + sb2_ref[...]
        out_ref[0] = jnp.maximum(outv, 0.0)


def kernel(x, tW1, tb1, tW2, tb2, tW3, tb3, sW1, sb1, sW2, sb2):
    xt = jnp.transpose(x, (0, 2, 1))

    eye = jnp.eye(_G, dtype=F32)
    tmp = tW2.reshape(_G, 4, 4, 3)
    W2blkT = jnp.einsum('gh,hoid->dgiho', eye, tmp).reshape(3, _W, _W)
    tW1T = tW1[:, :, 0].T
    tW3T = tW3[:, :, 0].T
    W1aT = sW1[:, :_C, 0, 0].T
    W1bT = sW1[:, _C:, 0, 0].T
    tmp2 = sW2.reshape(_G, 8, 4, 7, 7)
    W2T = jnp.einsum('gh,hoidw->dwgiho', eye, tmp2).reshape(7, 7, _W, _C)
    W2Tb = W2T.reshape(7, 7 * _W, _C).astype(BF16)
    tb1r = tb1.reshape(1, _W)
    tb2r = tb2.reshape(1, _W)
    tb3r = tb3.reshape(1, _C)
    sb1r = sb1.reshape(1, _W)
    sb2r = sb2.reshape(1, _C)

    outT = pl.pallas_call(
        _fused_kernel,
        grid=(_B, _NT + 1),
        in_specs=[
            pl.BlockSpec((1, _RT, _C), lambda b, r: (b, jnp.minimum(r, _NT - 1), 0)),
            pl.BlockSpec((1, _T, _C), lambda b, r: (b, 0, 0)),
            pl.BlockSpec((1, _C, _T), lambda b, r: (b, 0, 0)),
            pl.BlockSpec((_C, _W), lambda b, r: (0, 0)),
            pl.BlockSpec((1, _W), lambda b, r: (0, 0)),
            pl.BlockSpec((3, _W, _W), lambda b, r: (0, 0, 0)),
            pl.BlockSpec((1, _W), lambda b, r: (0, 0)),
            pl.BlockSpec((_W, _C), lambda b, r: (0, 0)),
            pl.BlockSpec((1, _C), lambda b, r: (0, 0)),
            pl.BlockSpec((_C, _W), lambda b, r: (0, 0)),
            pl.BlockSpec((_C, _W), lambda b, r: (0, 0)),
            pl.BlockSpec((1, _W), lambda b, r: (0, 0)),
            pl.BlockSpec((7, 7 * _W, _C), lambda b, r: (0, 0, 0)),
            pl.BlockSpec((1, _C), lambda b, r: (0, 0)),
        ],
        out_specs=pl.BlockSpec(
            (1, _RT, _C), lambda b, r: (b, jnp.maximum(r - 1, 0), 0)),
        out_shape=jax.ShapeDtypeStruct((_B, _T, _C), F32),
        scratch_shapes=[
            pltpu.VMEM((_PR, _NS * _W), BF16),
            pltpu.VMEM((_T, _C), F32),
            pltpu.VMEM((_T, _W), BF16),
            pltpu.VMEM((8, _T), F32),
        ],
    )(xt, xt, x, tW1T, tb1r, W2blkT, tb2r, tW3T, tb3r,
      W1aT, W1bT, sb1r, W2Tb, sb2r)

    return jnp.transpose(outT, (0, 2, 1))


# speedup vs baseline: 26.3185x; 1.0252x over previous
import jax
import jax.numpy as jnp
from jax.experimental import pallas as pl
from jax.experimental.pallas import tpu as pltpu

F32 = jnp.float32
BF16 = jnp.bfloat16
_B, _C, _T, _K, _G = 2, 256, 2048, 10, 32
_W = 128
_RT = 256
_NS = 16
_NT = _T // _RT
_PR = _T + 32
_DP = jax.lax.Precision.DEFAULT


def _mm(a, b):
    return jax.lax.dot_general(a, b, (((1,), (0,)), ((), ())),
                               precision=_DP, preferred_element_type=F32)


def _gelu(y):
    return 0.5 * y * (1.0 + jax.lax.erf(y * 0.7071067811865476))


def _fused_kernel(xt_tile_ref, xt_full_ref, x_ref,
                  tW1T_ref, tb1_ref, W2blkT_ref, tb2_ref, tW3T_ref, tb3_ref,
                  W1aT_ref, W1bT_ref, sb1_ref, W2Tb_ref, sb2_ref,
                  out_ref, pre_s, tout_s, at_s, xx_s):
    r = pl.program_id(1)

    @pl.when(r == 0)
    def _prep():
        xt = xt_full_ref[0]
        t1 = jnp.maximum(_mm(xt, tW1T_ref[...]) + tb1_ref[...], 0.0)
        z = jnp.zeros((1, _W), F32)
        t1m = jnp.concatenate([z, t1[:-1]], axis=0)
        t1p = jnp.concatenate([t1[1:], z], axis=0)
        t2 = (_mm(t1m, W2blkT_ref[0]) + _mm(t1, W2blkT_ref[1])
              + _mm(t1p, W2blkT_ref[2]))
        t2 = jnp.maximum(t2 + tb2_ref[...], 0.0)
        tout_s[...] = _mm(t2, tW3T_ref[...]) + tb3_ref[...]
        at_s[...] = _mm(xt, W1aT_ref[...]).astype(BF16)
        xb = x_ref[0]
        xx_s[0:1, :] = jnp.sum(xb * xb, axis=0, keepdims=True)
        zpad = jnp.zeros((16, _NS * _W), BF16)
        pre_s[0:16, :] = zpad
        pre_s[_PR - 16:_PR, :] = zpad

    @pl.when(r < _NT)
    def _knn():
        xt_t = xt_tile_ref[0]
        xt_f = xt_full_ref[0]
        g = jax.lax.dot_general(xt_t, xt_f, (((1,), (1,)), ((), ())),
                                precision=_DP, preferred_element_type=F32)
        xx_t = jnp.sum(xt_t * xt_t, axis=1, keepdims=True)
        d = 2.0 * g - xx_t - xx_s[0:1, :]
        iota = jax.lax.broadcasted_iota(jnp.int32, (_RT, _T), 1).astype(F32)
        rowid = (jnp.float32(r * _RT)
                 + jax.lax.broadcasted_iota(jnp.int32, (_RT, _T), 0).astype(F32))
        at = at_s[...]
        bv = _mm(xt_t, W1bT_ref[...]) + sb1_ref[...]
        row0 = pl.ds(r * _RT + 16, _RT)
        f0 = at_s[pl.ds(r * _RT, _RT), :].astype(F32)
        pre_s[row0, 3 * _W:4 * _W] = _gelu(f0 + bv).astype(BF16)
        d = jnp.where(iota == rowid, -jnp.inf, d)
        for i in range(1, _K):
            m = jnp.max(d, axis=1, keepdims=True)
            cand = jnp.where(d == m, iota, jnp.float32(_T))
            am = jnp.min(cand, axis=1, keepdims=True)
            oh = cand == am
            f = jax.lax.dot_general(oh.astype(BF16), at,
                                    (((1,), (0,)), ((), ())),
                                    preferred_element_type=F32)
            pre_s[row0, (3 + i) * _W:(4 + i) * _W] = _gelu(f + bv).astype(BF16)
            d = jnp.where(oh, -jnp.inf, d)
        zeros = jnp.zeros((_RT, _W), BF16)
        for s in (0, 1, 2, 13, 14, 15):
            pre_s[row0, s * _W:(s + 1) * _W] = zeros

    @pl.when(r > 0)
    def _conv():
        c = r - 1
        win = pre_s[pl.ds(c * _RT, _RT + 32), :]
        accs = [jnp.zeros((_RT, _C), F32) for _ in range(_K)]
        for dh in range(7):
            xin = win[13 + dh:13 + dh + _RT, :]
            for j in range(_K):
                lo = max(0, 3 - j)
                hi = min(7, 13 - j)
                accs[j] = accs[j] + jax.lax.dot_general(
                    xin[:, (j + lo) * _W:(j + hi) * _W],
                    W2Tb_ref[dh, lo * _W:hi * _W, :],
                    (((1,), (0,)), ((), ())),
                    preferred_element_type=F32)
        smax = accs[0]
        for j in range(1, _K):
            smax = jnp.maximum(smax, accs[j])
        tout_c = tout_s[pl.ds(c * _RT, _RT), :]
        xt_c = xt_full_ref[0, pl.ds(c * _RT, _RT), :]
        outv = tout_c + xt_c + smax + sb2_ref[...]
        out_ref[0] = jnp.maximum(outv, 0.0)


def kernel(x, tW1, tb1, tW2, tb2, tW3, tb3, sW1, sb1, sW2, sb2):
    xt = jnp.transpose(x, (0, 2, 1))

    eye = jnp.eye(_G, dtype=F32)
    tmp = tW2.reshape(_G, 4, 4, 3)
    W2blkT = jnp.einsum('gh,hoid->dgiho', eye, tmp).reshape(3, _W, _W)
    tW1T = tW1[:, :, 0].T
    tW3T = tW3[:, :, 0].T
    W1aT = sW1[:, :_C, 0, 0].T
    W1bT = sW1[:, _C:, 0, 0].T
    tmp2 = sW2.reshape(_G, 8, 4, 7, 7)
    W2T = jnp.einsum('gh,hoidw->dwgiho', eye, tmp2).reshape(7, 7, _W, _C)
    W2Tb = W2T.reshape(7, 7 * _W, _C).astype(BF16)
    tb1r = tb1.reshape(1, _W)
    tb2r = tb2.reshape(1, _W)
    tb3r = tb3.reshape(1, _C)
    sb1r = sb1.reshape(1, _W)
    sb2r = sb2.reshape(1, _C)

    outT = pl.pallas_call(
        _fused_kernel,
        grid=(_B, _NT + 1),
        in_specs=[
            pl.BlockSpec((1, _RT, _C), lambda b, r: (b, jnp.minimum(r, _NT - 1), 0)),
            pl.BlockSpec((1, _T, _C), lambda b, r: (b, 0, 0)),
            pl.BlockSpec((1, _C, _T), lambda b, r: (b, 0, 0)),
            pl.BlockSpec((_C, _W), lambda b, r: (0, 0)),
            pl.BlockSpec((1, _W), lambda b, r: (0, 0)),
            pl.BlockSpec((3, _W, _W), lambda b, r: (0, 0, 0)),
            pl.BlockSpec((1, _W), lambda b, r: (0, 0)),
            pl.BlockSpec((_W, _C), lambda b, r: (0, 0)),
            pl.BlockSpec((1, _C), lambda b, r: (0, 0)),
            pl.BlockSpec((_C, _W), lambda b, r: (0, 0)),
            pl.BlockSpec((_C, _W), lambda b, r: (0, 0)),
            pl.BlockSpec((1, _W), lambda b, r: (0, 0)),
            pl.BlockSpec((7, 7 * _W, _C), lambda b, r: (0, 0, 0)),
            pl.BlockSpec((1, _C), lambda b, r: (0, 0)),
        ],
        out_specs=pl.BlockSpec(
            (1, _RT, _C), lambda b, r: (b, jnp.maximum(r - 1, 0), 0)),
        out_shape=jax.ShapeDtypeStruct((_B, _T, _C), F32),
        compiler_params=pltpu.CompilerParams(
            dimension_semantics=("parallel", "arbitrary")),
        scratch_shapes=[
            pltpu.VMEM((_PR, _NS * _W), BF16),
            pltpu.VMEM((_T, _C), F32),
            pltpu.VMEM((_T, _W), BF16),
            pltpu.VMEM((8, _T), F32),
        ],
    )(xt, xt, x, tW1T, tb1r, W2blkT, tb2r, tW3T, tb3r,
      W1aT, W1bT, sb1r, W2Tb, sb2r)

    return jnp.transpose(outT, (0, 2, 1))


# speedup vs baseline: 26.8303x; 1.0194x over previous
import jax
import jax.numpy as jnp
from jax.experimental import pallas as pl
from jax.experimental.pallas import tpu as pltpu

F32 = jnp.float32
BF16 = jnp.bfloat16
_B, _C, _T, _K, _G = 2, 256, 2048, 10, 32
_W = 128
_RT = 256
_NS = 16
_NT = _T // _RT
_PR = _T + 32
_DP = jax.lax.Precision.DEFAULT


def _mm(a, b):
    return jax.lax.dot_general(a, b, (((1,), (0,)), ((), ())),
                               precision=_DP, preferred_element_type=F32)


def _gelu(y):
    return 0.5 * y * (1.0 + jax.lax.erf(y * 0.7071067811865476))


def _fused_kernel(xt_tile_ref, xt_full_ref, x_ref,
                  tW1T_ref, tb1_ref, W2blkT_ref, tb2_ref, tW3T_ref, tb3_ref,
                  W1aT_ref, W1bT_ref, sb1_ref, W2Tb_ref, sb2_ref,
                  out_ref, pre_s, tout_s, at_s, xx_s):
    r = pl.program_id(1)

    @pl.when(r == 0)
    def _prep():
        xt = xt_full_ref[0]
        t1 = jnp.maximum(_mm(xt, tW1T_ref[...]) + tb1_ref[...], 0.0)
        z = jnp.zeros((1, _W), F32)
        t1m = jnp.concatenate([z, t1[:-1]], axis=0)
        t1p = jnp.concatenate([t1[1:], z], axis=0)
        t2 = (_mm(t1m, W2blkT_ref[0]) + _mm(t1, W2blkT_ref[1])
              + _mm(t1p, W2blkT_ref[2]))
        t2 = jnp.maximum(t2 + tb2_ref[...], 0.0)
        tout_s[...] = _mm(t2, tW3T_ref[...]) + tb3_ref[...]
        at_s[...] = _mm(xt, W1aT_ref[...]).astype(BF16)
        xb = x_ref[0]
        xx_s[0:1, :] = jnp.sum(xb * xb, axis=0, keepdims=True)
        zpad = jnp.zeros((16, _NS * _W), BF16)
        pre_s[0:16, :] = zpad
        pre_s[_PR - 16:_PR, :] = zpad

    @pl.when(r < _NT)
    def _knn():
        xt_t = xt_tile_ref[0]
        xt_f = xt_full_ref[0]
        g = jax.lax.dot_general(xt_t, xt_f, (((1,), (1,)), ((), ())),
                                precision=_DP, preferred_element_type=F32)
        xx_t = jnp.sum(xt_t * xt_t, axis=1, keepdims=True)
        d = 2.0 * g - xx_t - xx_s[0:1, :]
        iota = jax.lax.broadcasted_iota(jnp.int32, (_RT, _T), 1).astype(F32)
        rowid = (jnp.float32(r * _RT)
                 + jax.lax.broadcasted_iota(jnp.int32, (_RT, _T), 0).astype(F32))
        at = at_s[...]
        bv = _mm(xt_t, W1bT_ref[...]) + sb1_ref[...]
        row0 = pl.ds(r * _RT + 16, _RT)
        f0 = at_s[pl.ds(r * _RT, _RT), :].astype(F32)
        pre_s[row0, 3 * _W:4 * _W] = _gelu(f0 + bv).astype(BF16)
        d = jnp.where(iota == rowid, -jnp.inf, d)
        for i in range(1, _K):
            am = jnp.argmax(d, axis=1)[:, None]
            oh = iota == am.astype(F32)
            f = jax.lax.dot_general(oh.astype(BF16), at,
                                    (((1,), (0,)), ((), ())),
                                    preferred_element_type=F32)
            pre_s[row0, (3 + i) * _W:(4 + i) * _W] = _gelu(f + bv).astype(BF16)
            d = jnp.where(oh, -jnp.inf, d)
        zeros = jnp.zeros((_RT, _W), BF16)
        for s in (0, 1, 2, 13, 14, 15):
            pre_s[row0, s * _W:(s + 1) * _W] = zeros

    @pl.when(r > 0)
    def _conv():
        c = r - 1
        win = pre_s[pl.ds(c * _RT, _RT + 32), :]
        accs = [jnp.zeros((_RT, _C), F32) for _ in range(_K)]
        for dh in range(7):
            xin = win[13 + dh:13 + dh + _RT, :]
            for j in range(_K):
                lo = max(0, 3 - j)
                hi = min(7, 13 - j)
                accs[j] = accs[j] + jax.lax.dot_general(
                    xin[:, (j + lo) * _W:(j + hi) * _W],
                    W2Tb_ref[dh, lo * _W:hi * _W, :],
                    (((1,), (0,)), ((), ())),
                    preferred_element_type=F32)
        smax = accs[0]
        for j in range(1, _K):
            smax = jnp.maximum(smax, accs[j])
        tout_c = tout_s[pl.ds(c * _RT, _RT), :]
        xt_c = xt_full_ref[0, pl.ds(c * _RT, _RT), :]
        outv = tout_c + xt_c + smax + sb2_ref[...]
        out_ref[0] = jnp.maximum(outv, 0.0)


def kernel(x, tW1, tb1, tW2, tb2, tW3, tb3, sW1, sb1, sW2, sb2):
    xt = jnp.transpose(x, (0, 2, 1))

    eye = jnp.eye(_G, dtype=F32)
    tmp = tW2.reshape(_G, 4, 4, 3)
    W2blkT = jnp.einsum('gh,hoid->dgiho', eye, tmp).reshape(3, _W, _W)
    tW1T = tW1[:, :, 0].T
    tW3T = tW3[:, :, 0].T
    W1aT = sW1[:, :_C, 0, 0].T
    W1bT = sW1[:, _C:, 0, 0].T
    tmp2 = sW2.reshape(_G, 8, 4, 7, 7)
    W2T = jnp.einsum('gh,hoidw->dwgiho', eye, tmp2).reshape(7, 7, _W, _C)
    W2Tb = W2T.reshape(7, 7 * _W, _C).astype(BF16)
    tb1r = tb1.reshape(1, _W)
    tb2r = tb2.reshape(1, _W)
    tb3r = tb3.reshape(1, _C)
    sb1r = sb1.reshape(1, _W)
    sb2r = sb2.reshape(1, _C)

    outT = pl.pallas_call(
        _fused_kernel,
        grid=(_B, _NT + 1),
        in_specs=[
            pl.BlockSpec((1, _RT, _C), lambda b, r: (b, jnp.minimum(r, _NT - 1), 0)),
            pl.BlockSpec((1, _T, _C), lambda b, r: (b, 0, 0)),
            pl.BlockSpec((1, _C, _T), lambda b, r: (b, 0, 0)),
            pl.BlockSpec((_C, _W), lambda b, r: (0, 0)),
            pl.BlockSpec((1, _W), lambda b, r: (0, 0)),
            pl.BlockSpec((3, _W, _W), lambda b, r: (0, 0, 0)),
            pl.BlockSpec((1, _W), lambda b, r: (0, 0)),
            pl.BlockSpec((_W, _C), lambda b, r: (0, 0)),
            pl.BlockSpec((1, _C), lambda b, r: (0, 0)),
            pl.BlockSpec((_C, _W), lambda b, r: (0, 0)),
            pl.BlockSpec((_C, _W), lambda b, r: (0, 0)),
            pl.BlockSpec((1, _W), lambda b, r: (0, 0)),
            pl.BlockSpec((7, 7 * _W, _C), lambda b, r: (0, 0, 0)),
            pl.BlockSpec((1, _C), lambda b, r: (0, 0)),
        ],
        out_specs=pl.BlockSpec(
            (1, _RT, _C), lambda b, r: (b, jnp.maximum(r - 1, 0), 0)),
        out_shape=jax.ShapeDtypeStruct((_B, _T, _C), F32),
        compiler_params=pltpu.CompilerParams(
            dimension_semantics=("parallel", "arbitrary")),
        scratch_shapes=[
            pltpu.VMEM((_PR, _NS * _W), BF16),
            pltpu.VMEM((_T, _C), F32),
            pltpu.VMEM((_T, _W), BF16),
            pltpu.VMEM((8, _T), F32),
        ],
    )(xt, xt, x, tW1T, tb1r, W2blkT, tb2r, tW3T, tb3r,
      W1aT, W1bT, sb1r, W2Tb, sb2r)

    return jnp.transpose(outT, (0, 2, 1))


# speedup vs baseline: 28.2056x; 1.0513x over previous
import jax
import jax.numpy as jnp
from jax.experimental import pallas as pl
from jax.experimental.pallas import tpu as pltpu

F32 = jnp.float32
BF16 = jnp.bfloat16
_B, _C, _T, _K, _G = 2, 256, 2048, 10, 32
_W = 128
_RT = 512
_NS = 16
_NT = _T // _RT
_PR = _T + 32
_DP = jax.lax.Precision.DEFAULT


def _mm(a, b):
    return jax.lax.dot_general(a, b, (((1,), (0,)), ((), ())),
                               precision=_DP, preferred_element_type=F32)


def _gelu(y):
    return 0.5 * y * (1.0 + jax.lax.erf(y * 0.7071067811865476))


def _fused_kernel(xt_tile_ref, xt_full_ref, x_ref,
                  tW1T_ref, tb1_ref, W2blkT_ref, tb2_ref, tW3T_ref, tb3_ref,
                  W1aT_ref, W1bT_ref, sb1_ref, W2Tb_ref, sb2_ref,
                  out_ref, pre_s, tout_s, at_s, xx_s):
    r = pl.program_id(1)

    @pl.when(r == 0)
    def _prep():
        xt = xt_full_ref[0]
        t1 = jnp.maximum(_mm(xt, tW1T_ref[...]) + tb1_ref[...], 0.0)
        z = jnp.zeros((1, _W), F32)
        t1m = jnp.concatenate([z, t1[:-1]], axis=0)
        t1p = jnp.concatenate([t1[1:], z], axis=0)
        t2 = (_mm(t1m, W2blkT_ref[0]) + _mm(t1, W2blkT_ref[1])
              + _mm(t1p, W2blkT_ref[2]))
        t2 = jnp.maximum(t2 + tb2_ref[...], 0.0)
        tout_s[...] = _mm(t2, tW3T_ref[...]) + tb3_ref[...]
        at_s[...] = _mm(xt, W1aT_ref[...]).astype(BF16)
        xb = x_ref[0]
        xx_s[0:1, :] = jnp.sum(xb * xb, axis=0, keepdims=True)
        zpad = jnp.zeros((16, _NS * _W), BF16)
        pre_s[0:16, :] = zpad
        pre_s[_PR - 16:_PR, :] = zpad

    @pl.when(r < _NT)
    def _knn():
        xt_t = xt_tile_ref[0]
        xt_f = xt_full_ref[0]
        g = jax.lax.dot_general(xt_t, xt_f, (((1,), (1,)), ((), ())),
                                precision=_DP, preferred_element_type=F32)
        xx_t = jnp.sum(xt_t * xt_t, axis=1, keepdims=True)
        d = 2.0 * g - xx_t - xx_s[0:1, :]
        iota = jax.lax.broadcasted_iota(jnp.int32, (_RT, _T), 1).astype(F32)
        rowid = (jnp.float32(r * _RT)
                 + jax.lax.broadcasted_iota(jnp.int32, (_RT, _T), 0).astype(F32))
        at = at_s[...]
        bv = _mm(xt_t, W1bT_ref[...]) + sb1_ref[...]
        row0 = pl.ds(r * _RT + 16, _RT)
        f0 = at_s[pl.ds(r * _RT, _RT), :].astype(F32)
        pre_s[row0, 3 * _W:4 * _W] = _gelu(f0 + bv).astype(BF16)
        d = jnp.where(iota == rowid, -jnp.inf, d)
        for i in range(1, _K):
            am = jnp.argmax(d, axis=1)[:, None]
            oh = iota == am.astype(F32)
            f = jax.lax.dot_general(oh.astype(BF16), at,
                                    (((1,), (0,)), ((), ())),
                                    preferred_element_type=F32)
            pre_s[row0, (3 + i) * _W:(4 + i) * _W] = _gelu(f + bv).astype(BF16)
            d = jnp.where(oh, -jnp.inf, d)
        zeros = jnp.zeros((_RT, _W), BF16)
        for s in (0, 1, 2, 13, 14, 15):
            pre_s[row0, s * _W:(s + 1) * _W] = zeros

    @pl.when(r > 0)
    def _conv():
        c = r - 1
        win = pre_s[pl.ds(c * _RT, _RT + 32), :]
        accs = [jnp.zeros((_RT, _C), F32) for _ in range(_K)]
        for dh in range(7):
            xin = win[13 + dh:13 + dh + _RT, :]
            for j in range(_K):
                lo = max(0, 3 - j)
                hi = min(7, 13 - j)
                accs[j] = accs[j] + jax.lax.dot_general(
                    xin[:, (j + lo) * _W:(j + hi) * _W],
                    W2Tb_ref[dh, lo * _W:hi * _W, :],
                    (((1,), (0,)), ((), ())),
                    preferred_element_type=F32)
        smax = accs[0]
        for j in range(1, _K):
            smax = jnp.maximum(smax, accs[j])
        tout_c = tout_s[pl.ds(c * _RT, _RT), :]
        xt_c = xt_full_ref[0, pl.ds(c * _RT, _RT), :]
        outv = tout_c + xt_c + smax + sb2_ref[...]
        out_ref[0] = jnp.maximum(outv, 0.0)


def kernel(x, tW1, tb1, tW2, tb2, tW3, tb3, sW1, sb1, sW2, sb2):
    xt = jnp.transpose(x, (0, 2, 1))

    eye = jnp.eye(_G, dtype=F32)
    tmp = tW2.reshape(_G, 4, 4, 3)
    W2blkT = jnp.einsum('gh,hoid->dgiho', eye, tmp).reshape(3, _W, _W)
    tW1T = tW1[:, :, 0].T
    tW3T = tW3[:, :, 0].T
    W1aT = sW1[:, :_C, 0, 0].T
    W1bT = sW1[:, _C:, 0, 0].T
    tmp2 = sW2.reshape(_G, 8, 4, 7, 7)
    W2T = jnp.einsum('gh,hoidw->dwgiho', eye, tmp2).reshape(7, 7, _W, _C)
    W2Tb = W2T.reshape(7, 7 * _W, _C).astype(BF16)
    tb1r = tb1.reshape(1, _W)
    tb2r = tb2.reshape(1, _W)
    tb3r = tb3.reshape(1, _C)
    sb1r = sb1.reshape(1, _W)
    sb2r = sb2.reshape(1, _C)

    outT = pl.pallas_call(
        _fused_kernel,
        grid=(_B, _NT + 1),
        in_specs=[
            pl.BlockSpec((1, _RT, _C), lambda b, r: (b, jnp.minimum(r, _NT - 1), 0)),
            pl.BlockSpec((1, _T, _C), lambda b, r: (b, 0, 0)),
            pl.BlockSpec((1, _C, _T), lambda b, r: (b, 0, 0)),
            pl.BlockSpec((_C, _W), lambda b, r: (0, 0)),
            pl.BlockSpec((1, _W), lambda b, r: (0, 0)),
            pl.BlockSpec((3, _W, _W), lambda b, r: (0, 0, 0)),
            pl.BlockSpec((1, _W), lambda b, r: (0, 0)),
            pl.BlockSpec((_W, _C), lambda b, r: (0, 0)),
            pl.BlockSpec((1, _C), lambda b, r: (0, 0)),
            pl.BlockSpec((_C, _W), lambda b, r: (0, 0)),
            pl.BlockSpec((_C, _W), lambda b, r: (0, 0)),
            pl.BlockSpec((1, _W), lambda b, r: (0, 0)),
            pl.BlockSpec((7, 7 * _W, _C), lambda b, r: (0, 0, 0)),
            pl.BlockSpec((1, _C), lambda b, r: (0, 0)),
        ],
        out_specs=pl.BlockSpec(
            (1, _RT, _C), lambda b, r: (b, jnp.maximum(r - 1, 0), 0)),
        out_shape=jax.ShapeDtypeStruct((_B, _T, _C), F32),
        compiler_params=pltpu.CompilerParams(
            dimension_semantics=("parallel", "arbitrary")),
        scratch_shapes=[
            pltpu.VMEM((_PR, _NS * _W), BF16),
            pltpu.VMEM((_T, _C), F32),
            pltpu.VMEM((_T, _W), BF16),
            pltpu.VMEM((8, _T), F32),
        ],
    )(xt, xt, x, tW1T, tb1r, W2blkT, tb2r, tW3T, tb3r,
      W1aT, W1bT, sb1r, W2Tb, sb2r)

    return jnp.transpose(outT, (0, 2, 1))


# speedup vs baseline: 28.6716x; 1.0165x over previous
import jax
import jax.numpy as jnp
from jax.experimental import pallas as pl
from jax.experimental.pallas import tpu as pltpu

F32 = jnp.float32
BF16 = jnp.bfloat16
_B, _C, _T, _K, _G = 2, 256, 2048, 10, 32
_W = 128
_RT = 512
_NS = 16
_NT = _T // _RT
_PR = _T + 32
_DP = jax.lax.Precision.DEFAULT


def _mm(a, b):
    return jax.lax.dot_general(a, b, (((1,), (0,)), ((), ())),
                               precision=_DP, preferred_element_type=F32)


def _gelu(y):
    return 0.5 * y * (1.0 + jax.lax.erf(y * 0.7071067811865476))


def _fused_kernel(xt_tile_ref, xt_full_ref, x_ref,
                  tW1T_ref, tb1_ref, W2blkT_ref, tb2_ref, tW3T_ref, tb3_ref,
                  W1aT_ref, W1bT_ref, sb1_ref, W2Tb_ref, sb2_ref,
                  out_ref, pre_s, tout_s, at_s, xx_s):
    r = pl.program_id(1)

    @pl.when(r == 0)
    def _prep():
        xt = xt_full_ref[0]
        t1 = jnp.maximum(_mm(xt, tW1T_ref[...]) + tb1_ref[...], 0.0)
        z = jnp.zeros((1, _W), F32)
        t1m = jnp.concatenate([z, t1[:-1]], axis=0)
        t1p = jnp.concatenate([t1[1:], z], axis=0)
        t2 = (_mm(t1m, W2blkT_ref[0]) + _mm(t1, W2blkT_ref[1])
              + _mm(t1p, W2blkT_ref[2]))
        t2 = jnp.maximum(t2 + tb2_ref[...], 0.0)
        tout_s[...] = _mm(t2, tW3T_ref[...]) + tb3_ref[...]
        at_s[...] = _mm(xt, W1aT_ref[...]).astype(BF16)
        xb = x_ref[0]
        xx_s[0:1, :] = jnp.sum(xb * xb, axis=0, keepdims=True)
        zpad = jnp.zeros((16, _NS * _W), BF16)
        pre_s[0:16, :] = zpad
        pre_s[_PR - 16:_PR, :] = zpad

    @pl.when(r < _NT)
    def _knn():
        xt_t = xt_tile_ref[0]
        xt_f = xt_full_ref[0]
        g = jax.lax.dot_general(xt_t, xt_f, (((1,), (1,)), ((), ())),
                                precision=_DP, preferred_element_type=F32)
        xx_t = jnp.sum(xt_t * xt_t, axis=1, keepdims=True)
        d = 2.0 * g - xx_t - xx_s[0:1, :]
        iota = jax.lax.broadcasted_iota(jnp.int32, (_RT, _T), 1).astype(F32)
        rowid = (jnp.float32(r * _RT)
                 + jax.lax.broadcasted_iota(jnp.int32, (_RT, _T), 0).astype(F32))
        at = at_s[...]
        bv = _mm(xt_t, W1bT_ref[...]) + sb1_ref[...]
        row0 = pl.ds(r * _RT + 16, _RT)
        f0 = at_s[pl.ds(r * _RT, _RT), :].astype(F32)
        pre_s[row0, 3 * _W:4 * _W] = _gelu(f0 + bv).astype(BF16)
        d = jnp.where(iota == rowid, -jnp.inf, d)
        for i in range(1, _K):
            am = jnp.argmax(d, axis=1)[:, None]
            oh = iota == am.astype(F32)
            f = jax.lax.dot_general(oh.astype(BF16), at,
                                    (((1,), (0,)), ((), ())),
                                    preferred_element_type=F32)
            pre_s[row0, (3 + i) * _W:(4 + i) * _W] = _gelu(f + bv).astype(BF16)
            d = jnp.where(oh, -jnp.inf, d)
        zeros = jnp.zeros((_RT, _W), BF16)
        for s in (0, 1, 2, 13, 14, 15):
            pre_s[row0, s * _W:(s + 1) * _W] = zeros

    @pl.when(r > 0)
    def _conv():
        c = r - 1
        win = pre_s[pl.ds(c * _RT, _RT + 32), :]
        accs = [jnp.zeros((_RT, _C), F32) for _ in range(_K)]
        for dh in range(7):
            xin = win[13 + dh:13 + dh + _RT, :]
            for j in range(_K):
                lo = max(0, 3 - j)
                hi = min(7, 13 - j)
                accs[j] = accs[j] + jax.lax.dot_general(
                    xin[:, (j + lo) * _W:(j + hi) * _W],
                    W2Tb_ref[dh, lo * _W:hi * _W, :],
                    (((1,), (0,)), ((), ())),
                    preferred_element_type=F32)
        smax = accs[0]
        for j in range(1, _K):
            smax = jnp.maximum(smax, accs[j])
        tout_c = tout_s[pl.ds(c * _RT, _RT), :]
        xt_c = xt_full_ref[0, pl.ds(c * _RT, _RT), :]
        outv = tout_c + xt_c + smax + sb2_ref[...]
        out_ref[0] = jnp.maximum(outv, 0.0).T


def kernel(x, tW1, tb1, tW2, tb2, tW3, tb3, sW1, sb1, sW2, sb2):
    xt = jnp.transpose(x, (0, 2, 1))

    eye = jnp.eye(_G, dtype=F32)
    tmp = tW2.reshape(_G, 4, 4, 3)
    W2blkT = jnp.einsum('gh,hoid->dgiho', eye, tmp).reshape(3, _W, _W)
    tW1T = tW1[:, :, 0].T
    tW3T = tW3[:, :, 0].T
    W1aT = sW1[:, :_C, 0, 0].T
    W1bT = sW1[:, _C:, 0, 0].T
    tmp2 = sW2.reshape(_G, 8, 4, 7, 7)
    W2T = jnp.einsum('gh,hoidw->dwgiho', eye, tmp2).reshape(7, 7, _W, _C)
    W2Tb = W2T.reshape(7, 7 * _W, _C).astype(BF16)
    tb1r = tb1.reshape(1, _W)
    tb2r = tb2.reshape(1, _W)
    tb3r = tb3.reshape(1, _C)
    sb1r = sb1.reshape(1, _W)
    sb2r = sb2.reshape(1, _C)

    out = pl.pallas_call(
        _fused_kernel,
        grid=(_B, _NT + 1),
        in_specs=[
            pl.BlockSpec((1, _RT, _C), lambda b, r: (b, jnp.minimum(r, _NT - 1), 0)),
            pl.BlockSpec((1, _T, _C), lambda b, r: (b, 0, 0)),
            pl.BlockSpec((1, _C, _T), lambda b, r: (b, 0, 0)),
            pl.BlockSpec((_C, _W), lambda b, r: (0, 0)),
            pl.BlockSpec((1, _W), lambda b, r: (0, 0)),
            pl.BlockSpec((3, _W, _W), lambda b, r: (0, 0, 0)),
            pl.BlockSpec((1, _W), lambda b, r: (0, 0)),
            pl.BlockSpec((_W, _C), lambda b, r: (0, 0)),
            pl.BlockSpec((1, _C), lambda b, r: (0, 0)),
            pl.BlockSpec((_C, _W), lambda b, r: (0, 0)),
            pl.BlockSpec((_C, _W), lambda b, r: (0, 0)),
            pl.BlockSpec((1, _W), lambda b, r: (0, 0)),
            pl.BlockSpec((7, 7 * _W, _C), lambda b, r: (0, 0, 0)),
            pl.BlockSpec((1, _C), lambda b, r: (0, 0)),
        ],
        out_specs=pl.BlockSpec(
            (1, _C, _RT), lambda b, r: (b, 0, jnp.maximum(r - 1, 0))),
        out_shape=jax.ShapeDtypeStruct((_B, _C, _T), F32),
        compiler_params=pltpu.CompilerParams(
            dimension_semantics=("parallel", "arbitrary")),
        scratch_shapes=[
            pltpu.VMEM((_PR, _NS * _W), BF16),
            pltpu.VMEM((_T, _C), F32),
            pltpu.VMEM((_T, _W), BF16),
            pltpu.VMEM((8, _T), F32),
        ],
    )(xt, xt, x, tW1T, tb1r, W2blkT, tb2r, tW3T, tb3r,
      W1aT, W1bT, sb1r, W2Tb, sb2r)

    return out


# speedup vs baseline: 29.1034x; 1.0151x over previous
import jax
import jax.numpy as jnp
from jax.experimental import pallas as pl
from jax.experimental.pallas import tpu as pltpu

F32 = jnp.float32
BF16 = jnp.bfloat16
_B, _C, _T, _K, _G = 2, 256, 2048, 10, 32
_W = 128
_RT = 512
_NS = 16
_NT = _T // _RT
_PR = _T + 32
_DP = jax.lax.Precision.DEFAULT


def _mm(a, b):
    return jax.lax.dot_general(a, b, (((1,), (0,)), ((), ())),
                               precision=_DP, preferred_element_type=F32)


def _gelu(y):
    return 0.5 * y * (1.0 + jax.lax.erf(y * 0.7071067811865476))


def _fused_kernel(x_ref,
                  tW1T_ref, tb1_ref, W2blkT_ref, tb2_ref, tW3T_ref, tb3_ref,
                  W1aT_ref, W1bT_ref, sb1_ref, W2Tb_ref, sb2_ref,
                  out_ref, pre_s, tout_s, at_s, xx_s, xt_s):
    r = pl.program_id(1)

    @pl.when(r == 0)
    def _prep():
        xb0 = x_ref[0]
        xt_s[...] = xb0.T
        xt = xt_s[...]
        t1 = jnp.maximum(_mm(xt, tW1T_ref[...]) + tb1_ref[...], 0.0)
        z = jnp.zeros((1, _W), F32)
        t1m = jnp.concatenate([z, t1[:-1]], axis=0)
        t1p = jnp.concatenate([t1[1:], z], axis=0)
        t2 = (_mm(t1m, W2blkT_ref[0]) + _mm(t1, W2blkT_ref[1])
              + _mm(t1p, W2blkT_ref[2]))
        t2 = jnp.maximum(t2 + tb2_ref[...], 0.0)
        tout_s[...] = _mm(t2, tW3T_ref[...]) + tb3_ref[...]
        at_s[...] = _mm(xt, W1aT_ref[...]).astype(BF16)
        xb = x_ref[0]
        xx_s[0:1, :] = jnp.sum(xb * xb, axis=0, keepdims=True)
        zpad = jnp.zeros((16, _NS * _W), BF16)
        pre_s[0:16, :] = zpad
        pre_s[_PR - 16:_PR, :] = zpad

    @pl.when(r < _NT)
    def _knn():
        xt_t = xt_s[pl.ds(r * _RT, _RT), :]
        xt_f = xt_s[...]
        g = jax.lax.dot_general(xt_t, xt_f, (((1,), (1,)), ((), ())),
                                precision=_DP, preferred_element_type=F32)
        xx_t = jnp.sum(xt_t * xt_t, axis=1, keepdims=True)
        d = 2.0 * g - xx_t - xx_s[0:1, :]
        iota = jax.lax.broadcasted_iota(jnp.int32, (_RT, _T), 1).astype(F32)
        rowid = (jnp.float32(r * _RT)
                 + jax.lax.broadcasted_iota(jnp.int32, (_RT, _T), 0).astype(F32))
        at = at_s[...]
        bv = _mm(xt_t, W1bT_ref[...]) + sb1_ref[...]
        row0 = pl.ds(r * _RT + 16, _RT)
        f0 = at_s[pl.ds(r * _RT, _RT), :].astype(F32)
        pre_s[row0, 3 * _W:4 * _W] = _gelu(f0 + bv).astype(BF16)
        d = jnp.where(iota == rowid, -jnp.inf, d)
        for i in range(1, _K):
            am = jnp.argmax(d, axis=1)[:, None]
            oh = iota == am.astype(F32)
            f = jax.lax.dot_general(oh.astype(BF16), at,
                                    (((1,), (0,)), ((), ())),
                                    preferred_element_type=F32)
            pre_s[row0, (3 + i) * _W:(4 + i) * _W] = _gelu(f + bv).astype(BF16)
            d = jnp.where(oh, -jnp.inf, d)
        zeros = jnp.zeros((_RT, _W), BF16)
        for s in (0, 1, 2, 13, 14, 15):
            pre_s[row0, s * _W:(s + 1) * _W] = zeros

    @pl.when(r > 0)
    def _conv():
        c = r - 1
        win = pre_s[pl.ds(c * _RT, _RT + 32), :]
        accs = [jnp.zeros((_RT, _C), F32) for _ in range(_K)]
        for dh in range(7):
            xin = win[13 + dh:13 + dh + _RT, :]
            for j in range(_K):
                lo = max(0, 3 - j)
                hi = min(7, 13 - j)
                accs[j] = accs[j] + jax.lax.dot_general(
                    xin[:, (j + lo) * _W:(j + hi) * _W],
                    W2Tb_ref[dh, lo * _W:hi * _W, :],
                    (((1,), (0,)), ((), ())),
                    preferred_element_type=F32)
        smax = accs[0]
        for j in range(1, _K):
            smax = jnp.maximum(smax, accs[j])
        tout_c = tout_s[pl.ds(c * _RT, _RT), :]
        xt_c = xt_s[pl.ds(c * _RT, _RT), :]
        outv = tout_c + xt_c + smax + sb2_ref[...]
        out_ref[0] = jnp.maximum(outv, 0.0).T


def kernel(x, tW1, tb1, tW2, tb2, tW3, tb3, sW1, sb1, sW2, sb2):
    eye = jnp.eye(_G, dtype=F32)
    tmp = tW2.reshape(_G, 4, 4, 3)
    W2blkT = jnp.einsum('gh,hoid->dgiho', eye, tmp).reshape(3, _W, _W)
    tW1T = tW1[:, :, 0].T
    tW3T = tW3[:, :, 0].T
    W1aT = sW1[:, :_C, 0, 0].T
    W1bT = sW1[:, _C:, 0, 0].T
    tmp2 = sW2.reshape(_G, 8, 4, 7, 7)
    W2T = jnp.einsum('gh,hoidw->dwgiho', eye, tmp2).reshape(7, 7, _W, _C)
    W2Tb = W2T.reshape(7, 7 * _W, _C).astype(BF16)
    tb1r = tb1.reshape(1, _W)
    tb2r = tb2.reshape(1, _W)
    tb3r = tb3.reshape(1, _C)
    sb1r = sb1.reshape(1, _W)
    sb2r = sb2.reshape(1, _C)

    out = pl.pallas_call(
        _fused_kernel,
        grid=(_B, _NT + 1),
        in_specs=[
            pl.BlockSpec((1, _C, _T), lambda b, r: (b, 0, 0)),
            pl.BlockSpec((_C, _W), lambda b, r: (0, 0)),
            pl.BlockSpec((1, _W), lambda b, r: (0, 0)),
            pl.BlockSpec((3, _W, _W), lambda b, r: (0, 0, 0)),
            pl.BlockSpec((1, _W), lambda b, r: (0, 0)),
            pl.BlockSpec((_W, _C), lambda b, r: (0, 0)),
            pl.BlockSpec((1, _C), lambda b, r: (0, 0)),
            pl.BlockSpec((_C, _W), lambda b, r: (0, 0)),
            pl.BlockSpec((_C, _W), lambda b, r: (0, 0)),
            pl.BlockSpec((1, _W), lambda b, r: (0, 0)),
            pl.BlockSpec((7, 7 * _W, _C), lambda b, r: (0, 0, 0)),
            pl.BlockSpec((1, _C), lambda b, r: (0, 0)),
        ],
        out_specs=pl.BlockSpec(
            (1, _C, _RT), lambda b, r: (b, 0, jnp.maximum(r - 1, 0))),
        out_shape=jax.ShapeDtypeStruct((_B, _C, _T), F32),
        compiler_params=pltpu.CompilerParams(
            dimension_semantics=("parallel", "arbitrary")),
        scratch_shapes=[
            pltpu.VMEM((_PR, _NS * _W), BF16),
            pltpu.VMEM((_T, _C), F32),
            pltpu.VMEM((_T, _W), BF16),
            pltpu.VMEM((8, _T), F32),
            pltpu.VMEM((_T, _C), F32),
        ],
    )(x, tW1T, tb1r, W2blkT, tb2r, tW3T, tb3r,
      W1aT, W1bT, sb1r, W2Tb, sb2r)

    return out
